# Initial kernel scaffold; baseline (speedup 1.0000x reference)
import jax
import jax.numpy as jnp
from jax.experimental import pallas as pl


def kernel(hidden_states, weight):
    raise NotImplementedError("write your pallas kernel here")


# speedup vs baseline: 1.0963x; 1.0963x over previous
import functools

import jax
import jax.numpy as jnp
from jax.experimental import pallas as pl
from jax.experimental.pallas import tpu as pltpu

TOPK = 8
N_EXP = 64


def _router_kernel(hs_ref, w_ref, probs_ref, vals_ref, idxs_ref):
    hs = hs_ref[...]
    w = w_ref[...]
    logits = jax.lax.dot_general(
        hs, w, (((1,), (1,)), ((), ())), preferred_element_type=jnp.float32
    )
    m = jnp.max(logits, axis=-1, keepdims=True)
    e = jnp.exp(logits - m)
    probs = e / jnp.sum(e, axis=-1, keepdims=True)
    probs_ref[...] = probs

    rows = probs.shape[0]
    lane = jax.lax.broadcasted_iota(jnp.int32, (rows, N_EXP), 1)
    work = probs
    for j in range(TOPK):
        vmax = jnp.max(work, axis=-1, keepdims=True)
        cand = jnp.where(work == vmax, lane, N_EXP)
        imin = jnp.min(cand, axis=-1, keepdims=True)
        vals_ref[:, j : j + 1] = vmax
        idxs_ref[:, j : j + 1] = imin
        work = jnp.where(lane == imin, -1.0, work)


@functools.partial(jax.jit, static_argnames=("block_rows",))
def kernel(hidden_states, weight, block_rows: int = 512):
    n_rows, d = hidden_states.shape
    n_exp = weight.shape[0]
    grid = (n_rows // block_rows,)
    probs, vals, idxs = pl.pallas_call(
        _router_kernel,
        grid=grid,
        in_specs=[
            pl.BlockSpec((block_rows, d), lambda i: (i, 0)),
            pl.BlockSpec((n_exp, d), lambda i: (0, 0)),
        ],
        out_specs=[
            pl.BlockSpec((block_rows, n_exp), lambda i: (i, 0)),
            pl.BlockSpec((block_rows, TOPK), lambda i: (i, 0)),
            pl.BlockSpec((block_rows, TOPK), lambda i: (i, 0)),
        ],
        out_shape=[
            jax.ShapeDtypeStruct((n_rows, n_exp), jnp.float32),
            jax.ShapeDtypeStruct((n_rows, TOPK), jnp.float32),
            jax.ShapeDtypeStruct((n_rows, TOPK), jnp.int32),
        ],
    )(hidden_states, weight)
    return (probs, vals, idxs)


# speedup vs baseline: 1.2221x; 1.1147x over previous
import functools

import jax
import jax.numpy as jnp
from jax.experimental import pallas as pl

TOPK = 8
N_EXP = 64


def _router_kernel(hs_ref, w_ref, probs_ref, vals_ref, idxs_ref):
    hs = hs_ref[...]
    w = w_ref[...]
    logits = jax.lax.dot_general(
        hs, w, (((1,), (1,)), ((), ())), preferred_element_type=jnp.float32
    )
    m = jnp.max(logits, axis=-1, keepdims=True)
    e = jnp.exp(logits - m)
    probs = e / jnp.sum(e, axis=-1, keepdims=True)
    probs_ref[...] = probs

    rows = probs.shape[0]
    lane = jax.lax.broadcasted_iota(jnp.int32, (rows, N_EXP), 1).astype(jnp.float32)
    vals_acc = jnp.zeros((rows, N_EXP), jnp.float32)
    idxs_acc = jnp.zeros((rows, N_EXP), jnp.float32)
    work = probs
    for j in range(TOPK):
        vmax = jnp.max(work, axis=-1, keepdims=True)
        cand = jnp.where(work == vmax, lane, float(N_EXP))
        imin = jnp.min(cand, axis=-1, keepdims=True)
        slot = lane == float(j)
        vals_acc = jnp.where(slot, vmax, vals_acc)
        idxs_acc = jnp.where(slot, imin, idxs_acc)
        work = jnp.where(cand == imin, -1.0, work)
    vals_ref[...] = vals_acc
    idxs_ref[...] = idxs_acc.astype(jnp.int32)


@functools.partial(jax.jit, static_argnames=("block_rows",))
def kernel(hidden_states, weight, block_rows: int = 512):
    n_rows, d = hidden_states.shape
    n_exp = weight.shape[0]
    grid = (n_rows // block_rows,)
    probs, vals, idxs = pl.pallas_call(
        _router_kernel,
        grid=grid,
        in_specs=[
            pl.BlockSpec((block_rows, d), lambda i: (i, 0)),
            pl.BlockSpec((n_exp, d), lambda i: (0, 0)),
        ],
        out_specs=[
            pl.BlockSpec((block_rows, n_exp), lambda i: (i, 0)),
            pl.BlockSpec((block_rows, n_exp), lambda i: (i, 0)),
            pl.BlockSpec((block_rows, n_exp), lambda i: (i, 0)),
        ],
        out_shape=[
            jax.ShapeDtypeStruct((n_rows, n_exp), jnp.float32),
            jax.ShapeDtypeStruct((n_rows, n_exp), jnp.float32),
            jax.ShapeDtypeStruct((n_rows, n_exp), jnp.int32),
        ],
    )(hidden_states, weight)
    return (probs, vals[:, :TOPK], idxs[:, :TOPK])
